# baseline (speedup 1.0000x reference)
import jax
import jax.numpy as jnp
from jax import lax
from jax.experimental import pallas as pl
from jax.experimental.pallas import tpu as pltpu
from jax.experimental.pallas import tpu_sc as plsc

BATCH = 16384
HIST = 50
EMBED = 32
HIDDEN = 64
VOCAB_ROWS = 1000000
PADH = 52
RPG = 4
IDXW = RPG * PADH
NC = 2
NS = 16
NW = NC * NS
BPW = BATCH // NW
NCH = BPW // RPG
NBUF = 4


def _sc_body(table_hbm, idx_hbm, out_hbm, idx_v, rows_v, sums_v, sems):
    cid = lax.axis_index("c")
    sid = lax.axis_index("s")
    wid = sid * NC + cid

    pltpu.sync_copy(idx_hbm.at[wid], idx_v)

    for b in range(NBUF):
        pltpu.async_copy(table_hbm.at[idx_v.at[b]], rows_v.at[b], sems.at[b])

    def step(i, carry):
        for b in range(NBUF):
            ch = i * NBUF + b
            pltpu.make_async_copy(
                table_hbm.at[idx_v.at[ch]], rows_v.at[b], sems.at[b]
            ).wait()
            for r in range(RPG):
                row = ch * RPG + r
                base = r * PADH
                a0 = rows_v[b, base, 0:16]
                a1 = rows_v[b, base, 16:32]
                for g in range(1, HIST):
                    a0 = a0 + rows_v[b, base + g, 0:16]
                    a1 = a1 + rows_v[b, base + g, 16:32]
                sums_v[row, 0:16] = a0
                sums_v[row, 16:32] = a1
            nxt = ch + NBUF

            @pl.when(nxt < NCH)
            def _():
                pltpu.async_copy(
                    table_hbm.at[idx_v.at[nxt]], rows_v.at[b], sems.at[b]
                )
        return carry

    lax.fori_loop(0, NCH // NBUF, step, 0)

    pltpu.sync_copy(sums_v, out_hbm.at[pl.ds(wid * BPW, BPW)])


_sc_gather_sum = pl.kernel(
    _sc_body,
    out_type=jax.ShapeDtypeStruct((BATCH, EMBED), jnp.float32),
    mesh=plsc.VectorSubcoreMesh(
        core_axis_name="c", subcore_axis_name="s", num_cores=NC, num_subcores=NS
    ),
    scratch_types=[
        pltpu.VMEM((NCH, IDXW), jnp.int32),
        pltpu.VMEM((NBUF, IDXW, EMBED), jnp.float32),
        pltpu.VMEM((BPW, EMBED), jnp.float32),
        pltpu.SemaphoreType.DMA((NBUF,)),
    ],
    compiler_params=pltpu.CompilerParams(use_tc_tiling_on_sc=False),
)


def _mlp_body(s_ref, w1_ref, b1_ref, w2_ref, b2_ref, o_ref):
    m = s_ref[...] * (1.0 / HIST)
    h = jnp.dot(m, w1_ref[...], preferred_element_type=jnp.float32)
    h = jnp.maximum(h + b1_ref[...], 0.0)
    z = jnp.dot(h, w2_ref[...], preferred_element_type=jnp.float32) + b2_ref[...]
    o_ref[...] = 1.0 / (1.0 + jnp.exp(-z))


def kernel(tag_indices, table, W1, b1, W2, b2):
    idx = jnp.pad(tag_indices.astype(jnp.int32), ((0, 0), (0, PADH - HIST)))
    idx = idx.reshape(NW, NCH, IDXW)
    table_lin = jax.lax.optimization_barrier(
        table.reshape(VOCAB_ROWS * EMBED // 128, 128)
    )
    sums = _sc_gather_sum(table_lin.reshape(VOCAB_ROWS, EMBED), idx)
    out = pl.pallas_call(
        _mlp_body,
        out_shape=jax.ShapeDtypeStruct((BATCH, 1), jnp.float32),
    )(sums, W1, b1.reshape(1, HIDDEN), W2, b2.reshape(1, 1))
    return out


# speedup vs baseline: 1.0714x; 1.0714x over previous
import jax
import jax.numpy as jnp
import numpy as np
from jax import lax
from jax.experimental import pallas as pl
from jax.experimental.pallas import tpu as pltpu
from jax.experimental.pallas import tpu_sc as plsc

BATCH = 16384
HIST = 50
EMBED = 32
HIDDEN = 64
VOCAB_ROWS = 1000000
PADH = 52
RPG = 4
IDXW = RPG * PADH
NC = 2
NS = 16
NW = NC * NS
BPW = BATCH // NW
NCH = BPW // RPG
NBUF = 4

_DEINT = np.concatenate([np.arange(0, EMBED, 2), np.arange(1, EMBED, 2)])


def _sc_body(table_hbm, idx_hbm, out_hbm, idx_v, rows_v, sums_v, sems):
    cid = lax.axis_index("c")
    sid = lax.axis_index("s")
    wid = sid * NC + cid

    pltpu.sync_copy(idx_hbm.at[wid], idx_v)

    for b in range(NBUF):
        pltpu.async_copy(table_hbm.at[idx_v.at[b]], rows_v.at[b], sems.at[b])

    def step(i, carry):
        for b in range(NBUF):
            ch = i * NBUF + b
            pltpu.make_async_copy(
                table_hbm.at[idx_v.at[ch]], rows_v.at[b], sems.at[b]
            ).wait()
            for r in range(RPG):
                row = ch * RPG + r
                base = r * PADH
                w = rows_v[b, base, 0:EMBED]
                a0, a1 = plsc.unpack(w, format=plsc.PackFormat.INTERLEAVED)
                for g in range(1, HIST):
                    w = rows_v[b, base + g, 0:EMBED]
                    lo, hi = plsc.unpack(w, format=plsc.PackFormat.INTERLEAVED)
                    a0 = a0 + lo
                    a1 = a1 + hi
                sums_v[row, 0:16] = a0
                sums_v[row, 16:32] = a1
            nxt = ch + NBUF

            @pl.when(nxt < NCH)
            def _():
                pltpu.async_copy(
                    table_hbm.at[idx_v.at[nxt]], rows_v.at[b], sems.at[b]
                )
        return carry

    lax.fori_loop(0, NCH // NBUF, step, 0)

    pltpu.sync_copy(sums_v, out_hbm.at[pl.ds(wid * BPW, BPW)])


_sc_gather_sum = pl.kernel(
    _sc_body,
    out_type=jax.ShapeDtypeStruct((BATCH, EMBED), jnp.float32),
    mesh=plsc.VectorSubcoreMesh(
        core_axis_name="c", subcore_axis_name="s", num_cores=NC, num_subcores=NS
    ),
    scratch_types=[
        pltpu.VMEM((NCH, IDXW), jnp.int32),
        pltpu.VMEM((NBUF, IDXW, EMBED), jnp.bfloat16),
        pltpu.VMEM((BPW, EMBED), jnp.float32),
        pltpu.SemaphoreType.DMA((NBUF,)),
    ],
    compiler_params=pltpu.CompilerParams(
        use_tc_tiling_on_sc=False, needs_layout_passes=False
    ),
)


def _mlp_body(s_ref, w1_ref, b1_ref, w2_ref, b2_ref, o_ref):
    m = s_ref[...] * (1.0 / HIST)
    h = jnp.dot(m, w1_ref[...], preferred_element_type=jnp.float32)
    h = jnp.maximum(h + b1_ref[...], 0.0)
    z = jnp.dot(h, w2_ref[...], preferred_element_type=jnp.float32) + b2_ref[...]
    o_ref[...] = 1.0 / (1.0 + jnp.exp(-z))


def kernel(tag_indices, table, W1, b1, W2, b2):
    idx = jnp.pad(tag_indices.astype(jnp.int32), ((0, 0), (0, PADH - HIST)))
    idx = idx.reshape(NW, NCH, IDXW)
    table_bf = table.astype(jnp.bfloat16)
    sums = _sc_gather_sum(table_bf, idx)
    w1_perm = W1[_DEINT, :]
    out = pl.pallas_call(
        _mlp_body,
        out_shape=jax.ShapeDtypeStruct((BATCH, 1), jnp.float32),
    )(sums, w1_perm, b1.reshape(1, HIDDEN), W2, b2.reshape(1, 1))
    return out
